# baseline (speedup 1.0000x reference)
import functools

import jax
import jax.numpy as jnp
from jax import lax
from jax.experimental import pallas as pl
from jax.experimental.pallas import tpu as pltpu
from jax.experimental.pallas import tpu_sc as plsc

N = 50000
CIN = 32
COUT = 32
G = 70
R = 343040
DUMP = 343000
NP = 50176
NPAD = 50008
KW = 9
NQ = NP * KW
NW = 32

BUILD_CHUNK = 1072
Q_CHUNK = 1008
P_CHUNK = Q_CHUNK // KW
BLKS = 1024
NBLK = R // BLKS

_mesh = plsc.VectorSubcoreMesh(core_axis_name="c", subcore_axis_name="s")
_sc_params = pltpu.CompilerParams(use_tc_tiling_on_sc=False)


@functools.partial(
    pl.kernel,
    out_type=jax.ShapeDtypeStruct((R, CIN), jnp.bfloat16),
    mesh=_mesh,
    compiler_params=_sc_params,
    scratch_types=[],
)
def _build_table(gridmin_hbm, feats_hbm, table_hbm):
    def body(i_vmem, o_vmem):
        pltpu.sync_copy(feats_hbm.at[i_vmem.at[0]], o_vmem)

    pltpu.emit_pipeline(
        body,
        grid=(R // BUILD_CHUNK,),
        in_specs=[pl.BlockSpec((1, BUILD_CHUNK), lambda i: (0, i))],
        out_specs=[pl.BlockSpec((BUILD_CHUNK, CIN), lambda i: (i, 0))],
        core_axis_name=("c", "s"),
        dimension_semantics=(pltpu.PARALLEL,),
    )(gridmin_hbm, table_hbm)


def _spass_body(prev_ref, cur_ref, nxt_ref, w_ref, o_ref):
    i = pl.program_id(0)
    cur = cur_ref[...]
    tm1 = jnp.concatenate([prev_ref[BLKS - 1:], cur[:-1]], axis=0)
    tp1 = jnp.concatenate([cur[1:], nxt_ref[:1]], axis=0)
    rid = lax.broadcasted_iota(jnp.int32, (BLKS, 1), 0)
    tm1 = jnp.where((i == 0) & (rid == 0), jnp.bfloat16(0), tm1)
    tp1 = jnp.where((i == NBLK - 1) & (rid == BLKS - 1), jnp.bfloat16(0), tp1)
    x = jnp.concatenate([tm1, cur, tp1], axis=1)
    y = jnp.dot(x, w_ref[...], preferred_element_type=jnp.float32)
    for p in range(KW):
        o_ref[p] = y[:, p * CIN:(p + 1) * CIN]


def _spass(table, w288):
    return pl.pallas_call(
        _spass_body,
        grid=(NBLK,),
        in_specs=[
            pl.BlockSpec((BLKS, CIN), lambda i: (jnp.maximum(i - 1, 0), 0)),
            pl.BlockSpec((BLKS, CIN), lambda i: (i, 0)),
            pl.BlockSpec((BLKS, CIN),
                         lambda i: (jnp.minimum(i + 1, NBLK - 1), 0)),
            pl.BlockSpec((3 * CIN, KW * COUT), lambda i: (0, 0)),
        ],
        out_specs=pl.BlockSpec((KW, BLKS, COUT), lambda i: (0, i, 0)),
        out_shape=jax.ShapeDtypeStruct((KW, R, COUT), jnp.float32),
    )(table, table, table, w288)


@functools.partial(
    pl.kernel,
    out_type=jax.ShapeDtypeStruct((NP, COUT), jnp.float32),
    mesh=_mesh,
    compiler_params=_sc_params,
    scratch_types=[pltpu.VMEM((Q_CHUNK, COUT), jnp.float32)],
)
def _gather_reduce(q_hbm, s_hbm, out_hbm, rows_v):
    def body(i_vmem, o_vmem):
        pltpu.sync_copy(s_hbm.at[i_vmem.at[0]], rows_v)

        @pl.loop(0, P_CHUNK)
        def _(r):
            for h in range(2):
                acc = rows_v[KW * r, pl.ds(16 * h, 16)]
                for m in range(1, KW):
                    acc = acc + rows_v[KW * r + m, pl.ds(16 * h, 16)]
                o_vmem[r, pl.ds(16 * h, 16)] = acc

    pltpu.emit_pipeline(
        body,
        grid=(NQ // Q_CHUNK,),
        in_specs=[pl.BlockSpec((1, Q_CHUNK), lambda i: (0, i))],
        out_specs=[pl.BlockSpec((P_CHUNK, COUT), lambda i: (i, 0))],
        core_axis_name=("c", "s"),
        dimension_semantics=(pltpu.PARALLEL,),
    )(q_hbm, out_hbm)


_OFFS9 = [(dx * G + dy) * G for dx in range(-1, 2) for dy in range(-1, 2)]


def kernel(feats, coords, kernel):
    w = kernel
    c = coords.astype(jnp.int32) + 1
    keys = (c[:, 0] * G + c[:, 1]) * G + c[:, 2]
    iota = jnp.arange(N, dtype=jnp.int32)
    gridmin = jnp.full((R,), N, jnp.int32).at[keys].min(iota)
    offs = jnp.array(_OFFS9, dtype=jnp.int32) + jnp.arange(KW, dtype=jnp.int32) * R
    q = keys[:, None] + offs[None, :]
    q = jnp.concatenate(
        [q, jnp.full((NP - N, 1), DUMP, jnp.int32)
         + jnp.arange(KW, dtype=jnp.int32)[None, :] * R], axis=0)
    q = q.reshape(NQ)
    feats_pad = jnp.concatenate(
        [feats.astype(jnp.bfloat16),
         jnp.zeros((NPAD - N, CIN), jnp.bfloat16)], axis=0)
    table = _build_table(gridmin.reshape(1, R), feats_pad)
    w288 = (w.reshape(KW, 3, CIN, COUT).transpose(1, 2, 0, 3)
            .reshape(3 * CIN, KW * COUT).astype(jnp.bfloat16))
    s = _spass(table, w288).reshape(KW * R, COUT)
    out = _gather_reduce(q.reshape(1, NQ), s)
    return out[:N]


# speedup vs baseline: 1.5770x; 1.5770x over previous
import functools

import jax
import jax.numpy as jnp
from jax import lax
from jax.experimental import pallas as pl
from jax.experimental.pallas import tpu as pltpu
from jax.experimental.pallas import tpu_sc as plsc

N = 50000
CIN = 32
COUT = 32
KV = 27
G = 70
R = 343040
DUMP = 343000
NP = 50176
NPAD = 50008
NQ = NP * KV
NW = 32

BUILD_CHUNK = 1072
Q_CHUNK = 1008
HALF = Q_CHUNK // 2

_mesh = plsc.VectorSubcoreMesh(core_axis_name="c", subcore_axis_name="s")
_sc_params = pltpu.CompilerParams(use_tc_tiling_on_sc=False)


@functools.partial(
    pl.kernel,
    out_type=jax.ShapeDtypeStruct((R, CIN), jnp.bfloat16),
    mesh=_mesh,
    compiler_params=_sc_params,
    scratch_types=[pltpu.SemaphoreType.DMA, pltpu.SemaphoreType.DMA],
)
def _build_table(gridmin_hbm, feats_hbm, table_hbm, sem0, sem1):
    def body(i_vmem, o_vmem):
        d0 = pltpu.async_copy(
            feats_hbm.at[i_vmem.at[0, pl.ds(0, BUILD_CHUNK // 2)]],
            o_vmem.at[pl.ds(0, BUILD_CHUNK // 2)], sem0)
        d1 = pltpu.async_copy(
            feats_hbm.at[i_vmem.at[0, pl.ds(BUILD_CHUNK // 2,
                                            BUILD_CHUNK // 2)]],
            o_vmem.at[pl.ds(BUILD_CHUNK // 2, BUILD_CHUNK // 2)], sem1)
        d0.wait()
        d1.wait()

    pltpu.emit_pipeline(
        body,
        grid=(R // BUILD_CHUNK,),
        in_specs=[pl.BlockSpec((1, BUILD_CHUNK), lambda i: (0, i))],
        out_specs=[pl.BlockSpec((BUILD_CHUNK, CIN), lambda i: (i, 0))],
        core_axis_name=("c", "s"),
        dimension_semantics=(pltpu.PARALLEL,),
    )(gridmin_hbm, table_hbm)


@functools.partial(
    pl.kernel,
    out_type=jax.ShapeDtypeStruct((NQ, CIN), jnp.bfloat16),
    mesh=_mesh,
    compiler_params=_sc_params,
    scratch_types=[pltpu.SemaphoreType.DMA, pltpu.SemaphoreType.DMA],
)
def _gather_rows(q_hbm, table_hbm, out_hbm, sem0, sem1):
    def body(i_vmem, o_vmem):
        d0 = pltpu.async_copy(
            table_hbm.at[i_vmem.at[0, pl.ds(0, HALF)]],
            o_vmem.at[pl.ds(0, HALF)], sem0)
        d1 = pltpu.async_copy(
            table_hbm.at[i_vmem.at[0, pl.ds(HALF, HALF)]],
            o_vmem.at[pl.ds(HALF, HALF)], sem1)
        d0.wait()
        d1.wait()

    pltpu.emit_pipeline(
        body,
        grid=(NQ // Q_CHUNK,),
        in_specs=[pl.BlockSpec((1, Q_CHUNK), lambda i: (0, i))],
        out_specs=[pl.BlockSpec((Q_CHUNK, CIN), lambda i: (i, 0))],
        core_axis_name=("c", "s"),
        dimension_semantics=(pltpu.PARALLEL,),
    )(q_hbm, out_hbm)


BLK = 512


def _mm_body(g_ref, w_ref, o_ref):
    o_ref[...] = jnp.dot(g_ref[...], w_ref[...],
                         preferred_element_type=jnp.float32)


def _matmul(gathered, wflat):
    return pl.pallas_call(
        _mm_body,
        grid=(NP // BLK,),
        in_specs=[
            pl.BlockSpec((BLK, KV * CIN), lambda i: (i, 0)),
            pl.BlockSpec((KV * CIN, COUT), lambda i: (0, 0)),
        ],
        out_specs=pl.BlockSpec((BLK, COUT), lambda i: (i, 0)),
        out_shape=jax.ShapeDtypeStruct((NP, COUT), jnp.float32),
    )(gathered, wflat)


_OFFS = [(dx * G + dy) * G + dz
         for dx in range(-1, 2) for dy in range(-1, 2) for dz in range(-1, 2)]


def kernel(feats, coords, kernel):
    w = kernel
    c = coords.astype(jnp.int32) + 1
    keys = (c[:, 0] * G + c[:, 1]) * G + c[:, 2]
    iota = jnp.arange(N, dtype=jnp.int32)
    gridmin = jnp.full((R,), N, jnp.int32).at[keys].min(iota)
    offs = jnp.array(_OFFS, dtype=jnp.int32)
    q = keys[:, None] + offs[None, :]
    q = jnp.concatenate([q, jnp.full((NP - N, KV), DUMP, jnp.int32)], axis=0)
    q = q.reshape(NQ)
    feats_pad = jnp.concatenate(
        [feats.astype(jnp.bfloat16),
         jnp.zeros((NPAD - N, CIN), jnp.bfloat16)], axis=0)
    table = _build_table(gridmin.reshape(1, R), feats_pad)
    gathered = _gather_rows(q.reshape(1, NQ), table)
    out = _matmul(gathered.reshape(NP, KV * CIN),
                  w.reshape(KV * CIN, COUT).astype(jnp.bfloat16))
    return out[:N]


# speedup vs baseline: 1.5795x; 1.0016x over previous
import functools

import jax
import jax.numpy as jnp
from jax import lax
from jax.experimental import pallas as pl
from jax.experimental.pallas import tpu as pltpu
from jax.experimental.pallas import tpu_sc as plsc

N = 50000
CIN = 32
COUT = 32
KV = 27
G = 70
R = 343040
DUMP = 343000
NP = 50176
NPAD = 50008
NQ = NP * KV
NSLAB = 2
NPS = NP // NSLAB
NQS = NPS * KV
NW = 32

BUILD_CHUNK = 1072
Q_CHUNK = 1008
HALF = Q_CHUNK // 2

_mesh = plsc.VectorSubcoreMesh(core_axis_name="c", subcore_axis_name="s")
_sc_params = pltpu.CompilerParams(use_tc_tiling_on_sc=False)


@functools.partial(
    pl.kernel,
    out_type=jax.ShapeDtypeStruct((R, CIN), jnp.bfloat16),
    mesh=_mesh,
    compiler_params=_sc_params,
    scratch_types=[pltpu.SemaphoreType.DMA, pltpu.SemaphoreType.DMA],
)
def _build_table(gridmin_hbm, feats_hbm, table_hbm, sem0, sem1):
    def body(i_vmem, o_vmem):
        d0 = pltpu.async_copy(
            feats_hbm.at[i_vmem.at[0, pl.ds(0, BUILD_CHUNK // 2)]],
            o_vmem.at[pl.ds(0, BUILD_CHUNK // 2)], sem0)
        d1 = pltpu.async_copy(
            feats_hbm.at[i_vmem.at[0, pl.ds(BUILD_CHUNK // 2,
                                            BUILD_CHUNK // 2)]],
            o_vmem.at[pl.ds(BUILD_CHUNK // 2, BUILD_CHUNK // 2)], sem1)
        d0.wait()
        d1.wait()

    pltpu.emit_pipeline(
        body,
        grid=(R // BUILD_CHUNK,),
        in_specs=[pl.BlockSpec((1, BUILD_CHUNK), lambda i: (0, i))],
        out_specs=[pl.BlockSpec((BUILD_CHUNK, CIN), lambda i: (i, 0))],
        core_axis_name=("c", "s"),
        dimension_semantics=(pltpu.PARALLEL,),
    )(gridmin_hbm, table_hbm)


@functools.partial(
    pl.kernel,
    out_type=jax.ShapeDtypeStruct((NQS, CIN), jnp.bfloat16),
    mesh=_mesh,
    compiler_params=_sc_params,
    scratch_types=[pltpu.SemaphoreType.DMA, pltpu.SemaphoreType.DMA],
)
def _gather_rows(q_hbm, table_hbm, out_hbm, sem0, sem1):
    def body(i_vmem, o_vmem):
        d0 = pltpu.async_copy(
            table_hbm.at[i_vmem.at[0, pl.ds(0, HALF)]],
            o_vmem.at[pl.ds(0, HALF)], sem0)
        d1 = pltpu.async_copy(
            table_hbm.at[i_vmem.at[0, pl.ds(HALF, HALF)]],
            o_vmem.at[pl.ds(HALF, HALF)], sem1)
        d0.wait()
        d1.wait()

    pltpu.emit_pipeline(
        body,
        grid=(NQS // Q_CHUNK,),
        in_specs=[pl.BlockSpec((1, Q_CHUNK), lambda i: (0, i))],
        out_specs=[pl.BlockSpec((Q_CHUNK, CIN), lambda i: (i, 0))],
        core_axis_name=("c", "s"),
        dimension_semantics=(pltpu.PARALLEL,),
    )(q_hbm, out_hbm)


BLK = 512


def _mm_body(g_ref, w_ref, o_ref):
    o_ref[...] = jnp.dot(g_ref[...], w_ref[...],
                         preferred_element_type=jnp.float32)


def _matmul(gathered, wflat):
    return pl.pallas_call(
        _mm_body,
        grid=(NPS // BLK,),
        in_specs=[
            pl.BlockSpec((BLK, KV * CIN), lambda i: (i, 0)),
            pl.BlockSpec((KV * CIN, COUT), lambda i: (0, 0)),
        ],
        out_specs=pl.BlockSpec((BLK, COUT), lambda i: (i, 0)),
        out_shape=jax.ShapeDtypeStruct((NPS, COUT), jnp.float32),
    )(gathered, wflat)


_OFFS = [(dx * G + dy) * G + dz
         for dx in range(-1, 2) for dy in range(-1, 2) for dz in range(-1, 2)]


def kernel(feats, coords, kernel):
    w = kernel
    c = coords.astype(jnp.int32) + 1
    keys = (c[:, 0] * G + c[:, 1]) * G + c[:, 2]
    iota = jnp.arange(N, dtype=jnp.int32)
    gridmin = jnp.full((R,), N, jnp.int32).at[keys].min(iota)
    offs = jnp.array(_OFFS, dtype=jnp.int32)
    q = keys[:, None] + offs[None, :]
    q = jnp.concatenate([q, jnp.full((NP - N, KV), DUMP, jnp.int32)], axis=0)
    q = q.reshape(NQ)
    feats_pad = jnp.concatenate(
        [feats.astype(jnp.bfloat16),
         jnp.zeros((NPAD - N, CIN), jnp.bfloat16)], axis=0)
    table = _build_table(gridmin.reshape(1, R), feats_pad)
    wflat = w.reshape(KV * CIN, COUT).astype(jnp.bfloat16)
    outs = []
    for sl in range(NSLAB):
        qs = lax.slice(q, (sl * NQS,), ((sl + 1) * NQS,)).reshape(1, NQS)
        gathered = _gather_rows(qs, table)
        outs.append(_matmul(gathered.reshape(NPS, KV * CIN), wflat))
    out = jnp.concatenate(outs, axis=0)
    return out[:N]
